# Initial kernel scaffold; baseline (speedup 1.0000x reference)
import jax
import jax.numpy as jnp
from jax.experimental import pallas as pl


def kernel(z, W):
    raise NotImplementedError("write your pallas kernel here")


# speedup vs baseline: 1.1177x; 1.1177x over previous
import functools

import jax
import jax.numpy as jnp
from jax import lax
from jax.experimental import pallas as pl
from jax.experimental.pallas import tpu as pltpu
from jax.experimental.pallas import tpu_sc as plsc

_N = 8192
_DIM = 32
_ROWS = 8192
_BLK = 1024
_K = 2048
_BETA = 0.25


def _bf16_rne(x):
    return x.astype(jnp.bfloat16).astype(jnp.float32)


def _argmin_body(z_ref, wt_ref, idx_ref, loss_ref):
    i = pl.program_id(0)
    zi = z_ref[...]
    zb = _bf16_rne(zi)
    z2 = jnp.sum(zi * zi, axis=1, keepdims=True)
    acc_bf = None
    for c in range(_N // _K):
        wt = wt_ref[:, c * _K:(c + 1) * _K]
        w2 = jnp.sum(wt * wt, axis=0, keepdims=True)
        zw = jnp.dot(zb, wt, preferred_element_type=jnp.float32)
        d = (z2 + w2) - 2.0 * zw
        m = jnp.min(d, axis=1, keepdims=True)
        col = lax.broadcasted_iota(jnp.int32, (_BLK, _K), 1)
        lidx = jnp.min(jnp.where(d == m, col, _N), axis=1, keepdims=True) + c * _K
        if acc_bf is None:
            acc_bf = _bf16_rne(m)
            acc_i = lidx
            acc_f = m
        else:
            lt = acc_bf < m
            keep = acc_bf <= m
            acc_i = jnp.where(keep, acc_i, lidx)
            acc_f = jnp.where(keep, acc_f, m)
            acc_bf = jnp.where(lt, acc_bf, _bf16_rne(m))
    idx_ref[...] = acc_i[:, 0]

    @pl.when(i == 0)
    def _():
        loss_ref[...] = jnp.zeros((1, 1), jnp.float32)

    loss_ref[...] = loss_ref[...] + jnp.sum(acc_f)

    @pl.when(i == pl.num_programs(0) - 1)
    def _():
        loss_ref[...] = loss_ref[...] * ((1.0 + _BETA) / float(_ROWS * _DIM))


_argmin = pl.pallas_call(
    _argmin_body,
    grid=(_ROWS // _BLK,),
    in_specs=[
        pl.BlockSpec((_BLK, _DIM), lambda i: (i, 0)),
        pl.BlockSpec((_DIM, _N), lambda i: (0, 0)),
    ],
    out_specs=[
        pl.BlockSpec((_BLK,), lambda i: (i,)),
        pl.BlockSpec((1, 1), lambda i: (0, 0)),
    ],
    out_shape=[
        jax.ShapeDtypeStruct((_ROWS,), jnp.int32),
        jax.ShapeDtypeStruct((1, 1), jnp.float32),
    ],
    compiler_params=pltpu.CompilerParams(dimension_semantics=("arbitrary",)),
)


def _sc_gather(table, idx2d):
    info = plsc.get_sparse_core_info()
    nc, ns = info.num_cores, info.num_subcores
    nw = nc * ns
    jcnt = idx2d.shape[0] // nw
    mesh = plsc.VectorSubcoreMesh(core_axis_name="c", subcore_axis_name="s")

    @functools.partial(
        pl.kernel,
        mesh=mesh,
        out_type=jax.ShapeDtypeStruct((_ROWS, _DIM), jnp.float32),
        scratch_types=[
            pltpu.VMEM((jcnt, 128), jnp.int32),
            pltpu.VMEM((128, _DIM), jnp.float32),
            pltpu.SemaphoreType.DMA,
        ],
        compiler_params=pltpu.CompilerParams(use_tc_tiling_on_sc=False),
    )
    def k(table_hbm, idx_hbm, out_hbm, idx_v, rows_v, sem):
        wid = lax.axis_index("s") * nc + lax.axis_index("c")
        pltpu.sync_copy(idx_hbm.at[pl.ds(wid * jcnt, jcnt)], idx_v)
        for j in range(jcnt):
            pltpu.async_copy(table_hbm.at[idx_v.at[j]], rows_v, sem).wait()
            pltpu.sync_copy(rows_v, out_hbm.at[pl.ds((wid * jcnt + j) * 128, 128)])

    return k(table, idx2d)


def kernel(z, W):
    b, l, c = z.shape
    zf = z.reshape(_ROWS, _DIM)
    idx, loss = _argmin(zf, W.T)
    z_q = _sc_gather(W, idx.reshape(-1, 128))
    return (z_q.reshape(b, l, c), idx.reshape(b, l, 1), loss[0, 0])


# speedup vs baseline: 1.2740x; 1.1398x over previous
import functools

import jax
import jax.numpy as jnp
from jax import lax
from jax.experimental import pallas as pl
from jax.experimental.pallas import tpu as pltpu
from jax.experimental.pallas import tpu_sc as plsc

_N = 8192
_DIM = 32
_ROWS = 8192
_BLK = 1024
_K = 2048
_BETA = 0.25


def _bf16_rne(x):
    return x.astype(jnp.bfloat16).astype(jnp.float32)


def _argmin_body(z_ref, wt_ref, idx_ref, loss_ref):
    i = pl.program_id(0)
    zi = z_ref[...]
    zb = _bf16_rne(zi)
    z2 = jnp.sum(zi * zi, axis=1, keepdims=True)
    colf = lax.broadcasted_iota(jnp.int32, (_BLK, _K), 1).astype(jnp.float32)
    acc_bf = None
    for c in range(_N // _K):
        wt = wt_ref[:, c * _K:(c + 1) * _K]
        w2 = jnp.sum(wt * wt, axis=0, keepdims=True)
        zw2 = jnp.dot(zb, wt + wt, preferred_element_type=jnp.float32)
        d = (z2 + w2) - zw2
        m = jnp.min(d, axis=1, keepdims=True)
        lidx_f = jnp.min(jnp.where(d == m, colf, float(_K)), axis=1, keepdims=True)
        lidx = lidx_f.astype(jnp.int32) + c * _K
        if acc_bf is None:
            acc_bf = _bf16_rne(m)
            acc_i = lidx
            acc_f = m
        else:
            lt = acc_bf < m
            keep = acc_bf <= m
            acc_i = jnp.where(keep, acc_i, lidx)
            acc_f = jnp.where(keep, acc_f, m)
            acc_bf = jnp.where(lt, acc_bf, _bf16_rne(m))
    idx_ref[...] = acc_i[:, 0]

    @pl.when(i == 0)
    def _():
        loss_ref[...] = jnp.zeros((1, 1), jnp.float32)

    loss_ref[...] = loss_ref[...] + jnp.sum(acc_f)

    @pl.when(i == pl.num_programs(0) - 1)
    def _():
        loss_ref[...] = loss_ref[...] * ((1.0 + _BETA) / float(_ROWS * _DIM))


_argmin = pl.pallas_call(
    _argmin_body,
    grid=(_ROWS // _BLK,),
    in_specs=[
        pl.BlockSpec((_BLK, _DIM), lambda i: (i, 0)),
        pl.BlockSpec((_DIM, _N), lambda i: (0, 0)),
    ],
    out_specs=[
        pl.BlockSpec((_BLK,), lambda i: (i,)),
        pl.BlockSpec((1, 1), lambda i: (0, 0)),
    ],
    out_shape=[
        jax.ShapeDtypeStruct((_ROWS,), jnp.int32),
        jax.ShapeDtypeStruct((1, 1), jnp.float32),
    ],
    compiler_params=pltpu.CompilerParams(dimension_semantics=("arbitrary",)),
)


def _sc_gather(table, idx2d):
    info = plsc.get_sparse_core_info()
    nc, ns = info.num_cores, info.num_subcores
    nw = nc * ns
    jcnt = idx2d.shape[0] // nw
    mesh = plsc.VectorSubcoreMesh(core_axis_name="c", subcore_axis_name="s")

    @functools.partial(
        pl.kernel,
        mesh=mesh,
        out_type=jax.ShapeDtypeStruct((_ROWS, _DIM), jnp.float32),
        scratch_types=[
            pltpu.VMEM((jcnt, 128), jnp.int32),
            pltpu.VMEM((128, _DIM), jnp.float32),
            pltpu.SemaphoreType.DMA,
        ],
        compiler_params=pltpu.CompilerParams(use_tc_tiling_on_sc=False),
    )
    def k(table_hbm, idx_hbm, out_hbm, idx_v, rows_v, sem):
        wid = lax.axis_index("s") * nc + lax.axis_index("c")
        pltpu.sync_copy(idx_hbm.at[pl.ds(wid * jcnt, jcnt)], idx_v)
        for j in range(jcnt):
            pltpu.async_copy(table_hbm.at[idx_v.at[j]], rows_v, sem).wait()
            pltpu.sync_copy(rows_v, out_hbm.at[pl.ds((wid * jcnt + j) * 128, 128)])

    return k(table, idx2d)


def kernel(z, W):
    b, l, c = z.shape
    zf = z.reshape(_ROWS, _DIM)
    idx, loss = _argmin(zf, W.T)
    z_q = _sc_gather(W, idx.reshape(-1, 128))
    return (z_q.reshape(b, l, c), idx.reshape(b, l, 1), loss[0, 0])


# speedup vs baseline: 1.2772x; 1.0025x over previous
import functools

import jax
import jax.numpy as jnp
from jax import lax
from jax.experimental import pallas as pl
from jax.experimental.pallas import tpu as pltpu
from jax.experimental.pallas import tpu_sc as plsc

_N = 8192
_DIM = 32
_ROWS = 8192
_BLK = 2048
_K = 2048
_BETA = 0.25


def _bf16_rne(x):
    return x.astype(jnp.bfloat16).astype(jnp.float32)


def _argmin_body(z_ref, wt_ref, idx_ref, loss_ref):
    i = pl.program_id(0)
    zi = z_ref[...]
    zb = _bf16_rne(zi)
    z2 = jnp.sum(zi * zi, axis=1, keepdims=True)
    colf = lax.broadcasted_iota(jnp.int32, (_BLK, _K), 1).astype(jnp.float32)
    acc_bf = None
    for c in range(_N // _K):
        wt = wt_ref[:, c * _K:(c + 1) * _K]
        w2 = jnp.sum(wt * wt, axis=0, keepdims=True)
        zw2 = jnp.dot(zb, wt + wt, preferred_element_type=jnp.float32)
        d = (z2 + w2) - zw2
        m = jnp.min(d, axis=1, keepdims=True)
        lidx_f = jnp.min(jnp.where(d == m, colf, float(_K)), axis=1, keepdims=True)
        lidx = lidx_f.astype(jnp.int32) + c * _K
        if acc_bf is None:
            acc_bf = _bf16_rne(m)
            acc_i = lidx
            acc_f = m
        else:
            lt = acc_bf < m
            keep = acc_bf <= m
            acc_i = jnp.where(keep, acc_i, lidx)
            acc_f = jnp.where(keep, acc_f, m)
            acc_bf = jnp.where(lt, acc_bf, _bf16_rne(m))
    idx_ref[...] = acc_i[:, 0]

    @pl.when(i == 0)
    def _():
        loss_ref[...] = jnp.zeros((1, 1), jnp.float32)

    loss_ref[...] = loss_ref[...] + jnp.sum(acc_f)

    @pl.when(i == pl.num_programs(0) - 1)
    def _():
        loss_ref[...] = loss_ref[...] * ((1.0 + _BETA) / float(_ROWS * _DIM))


_argmin = pl.pallas_call(
    _argmin_body,
    grid=(_ROWS // _BLK,),
    in_specs=[
        pl.BlockSpec((_BLK, _DIM), lambda i: (i, 0)),
        pl.BlockSpec((_DIM, _N), lambda i: (0, 0)),
    ],
    out_specs=[
        pl.BlockSpec((_BLK,), lambda i: (i,)),
        pl.BlockSpec((1, 1), lambda i: (0, 0)),
    ],
    out_shape=[
        jax.ShapeDtypeStruct((_ROWS,), jnp.int32),
        jax.ShapeDtypeStruct((1, 1), jnp.float32),
    ],
    compiler_params=pltpu.CompilerParams(dimension_semantics=("arbitrary",)),
)


def _sc_gather(table, idx2d):
    info = plsc.get_sparse_core_info()
    nc, ns = info.num_cores, info.num_subcores
    nw = nc * ns
    jcnt = idx2d.shape[0] // nw
    mesh = plsc.VectorSubcoreMesh(core_axis_name="c", subcore_axis_name="s")

    @functools.partial(
        pl.kernel,
        mesh=mesh,
        out_type=jax.ShapeDtypeStruct((_ROWS, _DIM), jnp.float32),
        scratch_types=[
            pltpu.VMEM((jcnt, 128), jnp.int32),
            pltpu.VMEM((128, _DIM), jnp.float32),
            pltpu.SemaphoreType.DMA,
        ],
        compiler_params=pltpu.CompilerParams(use_tc_tiling_on_sc=False),
    )
    def k(table_hbm, idx_hbm, out_hbm, idx_v, rows_v, sem):
        wid = lax.axis_index("s") * nc + lax.axis_index("c")
        pltpu.sync_copy(idx_hbm.at[pl.ds(wid * jcnt, jcnt)], idx_v)
        for j in range(jcnt):
            pltpu.async_copy(table_hbm.at[idx_v.at[j]], rows_v, sem).wait()
            pltpu.sync_copy(rows_v, out_hbm.at[pl.ds((wid * jcnt + j) * 128, 128)])

    return k(table, idx2d)


def kernel(z, W):
    b, l, c = z.shape
    zf = z.reshape(_ROWS, _DIM)
    idx, loss = _argmin(zf, W.T)
    z_q = _sc_gather(W, idx.reshape(-1, 128))
    return (z_q.reshape(b, l, c), idx.reshape(b, l, 1), loss[0, 0])
